# baseline (speedup 1.0000x reference)
import jax
import jax.numpy as jnp
from jax import lax
from jax.experimental import pallas as pl
from jax.experimental.pallas import tpu as pltpu
from jax.experimental.pallas import tpu_sc as plsc

D = 32
B = 16384
NC = 2
NS = 16
NW = NC * NS
BPW = B // NW
L = 16
CHUNK = 128
NCHUNK = BPW // CHUNK
UNROLL = 4


def _sc_body(uid_hbm, iid_hbm, ut_hbm, it_hbm, ubt_hbm, ibt_hbm, gb_hbm,
             out_hbm,
             idx_u, idx_i, u_rows, i_rows, ub, ib, gb, out_v, sem):
    wid = lax.axis_index("s") * NC + lax.axis_index("c")
    base = wid * BPW

    pltpu.sync_copy(uid_hbm.at[pl.ds(base, BPW)], idx_u)
    pltpu.sync_copy(iid_hbm.at[pl.ds(base, BPW)], idx_i)
    pltpu.sync_copy(gb_hbm, gb.at[pl.ds(0, 1)])

    bias_copies = []
    for c in range(NCHUNK):
        sl = pl.ds(c * CHUNK, CHUNK)
        bias_copies.append(
            pltpu.async_copy(ubt_hbm.at[idx_u.at[sl]], ub.at[sl], sem))
        bias_copies.append(
            pltpu.async_copy(ibt_hbm.at[idx_i.at[sl]], ib.at[sl], sem))

    def issue(jj, carry):
        ublk = idx_u[pl.ds(jj * L, L)] * D
        iblk = idx_i[pl.ds(jj * L, L)] * D
        for k in range(L):
            j = jj * L + k
            uoff = pl.multiple_of(ublk[k], D)
            ioff = pl.multiple_of(iblk[k], D)
            pltpu.async_copy(ut_hbm.at[pl.ds(uoff, D)],
                             u_rows.at[pl.ds(j * D, D)], sem)
            pltpu.async_copy(it_hbm.at[pl.ds(ioff, D)],
                             i_rows.at[pl.ds(j * D, D)], sem)
        return carry

    lax.fori_loop(0, BPW // L, issue, 0)

    for cp in bias_copies:
        cp.wait()
    pltpu.make_async_copy(ut_hbm.at[pl.ds(0, BPW * D)], u_rows, sem).wait()
    pltpu.make_async_copy(it_hbm.at[pl.ds(0, BPW * D)], i_rows, sem).wait()

    iota = lax.broadcasted_iota(jnp.int32, (L,), 0)
    gbias = gb[pl.ds(0, L)][0]

    def group(g, carry):
        flat0 = (iota + g * L) * D
        acc = ub[pl.ds(g * L, L)] + ib[pl.ds(g * L, L)] + gbias
        for d in range(D):
            uvec = plsc.load_gather(u_rows, [flat0 + d])
            ivec = plsc.load_gather(i_rows, [flat0 + d])
            acc = acc + uvec * ivec
        out_v[pl.ds(g * L, L)] = acc
        return carry

    lax.fori_loop(0, BPW // L, group, 0)

    pltpu.sync_copy(out_v, out_hbm.at[pl.ds(base, BPW)])


@jax.jit
def kernel(user_id, item_id, user_table, item_table, user_bias_table,
           item_bias_table, global_bias):
    user_id = user_id.astype(jnp.int32)
    item_id = item_id.astype(jnp.int32)
    ut_lin = user_table.reshape(-1)
    it_lin = item_table.reshape(-1)
    ubt = user_bias_table.reshape(-1)
    ibt = item_bias_table.reshape(-1)
    mesh = plsc.VectorSubcoreMesh(core_axis_name="c", subcore_axis_name="s")
    f = pl.kernel(
        _sc_body,
        out_type=jax.ShapeDtypeStruct((B,), jnp.float32),
        mesh=mesh,
        scratch_types=[
            pltpu.VMEM((BPW,), jnp.int32),
            pltpu.VMEM((BPW,), jnp.int32),
            pltpu.VMEM((BPW * D,), jnp.float32),
            pltpu.VMEM((BPW * D,), jnp.float32),
            pltpu.VMEM((BPW,), jnp.float32),
            pltpu.VMEM((BPW,), jnp.float32),
            pltpu.VMEM((L,), jnp.float32),
            pltpu.VMEM((BPW,), jnp.float32),
            pltpu.SemaphoreType.DMA,
        ],
        compiler_params=pltpu.CompilerParams(
            needs_layout_passes=False, use_tc_tiling_on_sc=False),
    )
    return f(user_id, item_id, ut_lin, it_lin, ubt, ibt, global_bias)
